# Initial kernel scaffold; baseline (speedup 1.0000x reference)
import jax
import jax.numpy as jnp
from jax.experimental import pallas as pl


def kernel(x, edge_index, batch, W1, b1, W2, b2, W3, b3, Wlin, blin):
    raise NotImplementedError("write your pallas kernel here")


# speedup vs baseline: 6.5350x; 6.5350x over previous
import functools

import jax
import jax.numpy as jnp
from jax import lax
from jax.experimental import pallas as pl
from jax.experimental.pallas import tpu as pltpu
from jax.experimental.pallas import tpu_sc as plsc

_N = 10000
_E = 160000
_DIN = 256
_H = 512
_G = 64
_DOUT = 500

_NP = 10240
_PE = 163840
_NS = 16
_NC = 2
_EB = _PE // (_NS * 128)
_DB = _PE // (_NC * _NS * 128)
_RT = _NP // _NS
_BR = 512
_NBLK = _NP // _BR


def _sc_mesh():
    return plsc.VectorSubcoreMesh(core_axis_name="c", subcore_axis_name="s")


def _deg_call(dst3, zeros16, ones16):
    @functools.partial(
        pl.kernel,
        out_type=jax.ShapeDtypeStruct((_NC, _NP, 16), jnp.float32),
        mesh=_sc_mesh(),
        scratch_types=[
            pltpu.VMEM((_DB, 128), jnp.int32),
            pltpu.VMEM((128, 16), jnp.float32),
            pltpu.VMEM_SHARED((_NP, 16), jnp.float32),
        ],
    )
    def deg_kernel(dst_hbm, z_hbm, o_hbm, out_hbm, idx_v, ones_v, acc):
        c = lax.axis_index("c")
        s = lax.axis_index("s")
        wid = c * _NS + s
        pltpu.sync_copy(dst_hbm.at[wid], idx_v)
        pltpu.sync_copy(o_hbm, ones_v)
        pltpu.sync_copy(z_hbm.at[pl.ds(s * _RT, _RT)], acc.at[pl.ds(s * _RT, _RT)])
        plsc.subcore_barrier()

        def step(j, carry):
            pltpu.sync_copy(ones_v, acc.at[idx_v.at[j]], add=True)
            return carry

        lax.fori_loop(0, _DB, step, 0)
        plsc.subcore_barrier()
        pltpu.sync_copy(acc.at[pl.ds(s * _RT, _RT)],
                        out_hbm.at[c, pl.ds(s * _RT, _RT)])

    return deg_kernel(dst3, zeros16, ones16)


def _scatter_call(z_chunks, src3, dst3s):
    nc = len(z_chunks)
    nc_half = nc // 2

    @functools.partial(
        pl.kernel,
        out_type=[jax.ShapeDtypeStruct((_NP, 128), jnp.float32)] * nc,
        mesh=_sc_mesh(),
        scratch_types=[
            pltpu.VMEM((_EB, 128), jnp.int32),
            pltpu.VMEM((_EB, 128), jnp.int32),
            pltpu.VMEM((128, 128), jnp.float32),
            pltpu.VMEM_SHARED((_NP, 128), jnp.float32),
            pltpu.SemaphoreType.DMA,
        ],
    )
    def scat_kernel(*refs):
        z_refs = refs[:nc]
        src_hbm = refs[nc]
        dst_hbm = refs[nc + 1]
        y_refs = refs[nc + 2:2 * nc + 2]
        src_idx, dst_idx, gbuf, acc, sem = refs[2 * nc + 2:]
        c = lax.axis_index("c")
        s = lax.axis_index("s")
        pltpu.sync_copy(src_hbm.at[s], src_idx)
        pltpu.sync_copy(dst_hbm.at[s], dst_idx)

        for chunk in range(nc):
            z_ref = z_refs[chunk]
            y_ref = y_refs[chunk]

            @pl.when(c == chunk // nc_half)
            def _():
                pltpu.sync_copy(z_ref.at[pl.ds(s * _RT, _RT)],
                                acc.at[pl.ds(s * _RT, _RT)])
                plsc.subcore_barrier()

                def step(j, carry):
                    pltpu.async_copy(z_ref.at[src_idx.at[j]], gbuf, sem).wait()
                    pltpu.sync_copy(gbuf, acc.at[dst_idx.at[j]], add=True)
                    return carry

                lax.fori_loop(0, _EB, step, 0)
                plsc.subcore_barrier()
                pltpu.sync_copy(acc.at[pl.ds(s * _RT, _RT)],
                                y_ref.at[pl.ds(s * _RT, _RT)])
                plsc.subcore_barrier()

    return scat_kernel(*z_chunks, src3, dst3s)


def _dinv_from(deg_blk):
    cnt = deg_blk[0, :, 0:1] + deg_blk[1, :, 0:1]
    return lax.rsqrt(1.0 + cnt)


_DEG_SPEC = pl.BlockSpec((_NC, _BR, 16), lambda i: (0, i, 0))


def _full_spec(shape):
    return pl.BlockSpec(shape, lambda i: tuple(0 for _ in shape))


def _prescale_call(x_pad, deg2):
    def body(x_ref, deg_ref, o0_ref, o1_ref):
        z = x_ref[...] * _dinv_from(deg_ref[...])
        o0_ref[...] = z[:, :128]
        o1_ref[...] = z[:, 128:]

    return pl.pallas_call(
        body,
        grid=(_NBLK,),
        in_specs=[pl.BlockSpec((_BR, _DIN), lambda i: (i, 0)), _DEG_SPEC],
        out_specs=[pl.BlockSpec((_BR, 128), lambda i: (i, 0))] * 2,
        out_shape=[jax.ShapeDtypeStruct((_NP, 128), jnp.float32)] * 2,
    )(x_pad, deg2)


def _mm12_call(y0, deg2, W1, b1, W2):
    def body(yc0, yc1, deg_ref, w1_ref, b1_ref, w2_ref, *outs):
        dinv = _dinv_from(deg_ref[...])
        a = jnp.concatenate([yc0[...], yc1[...]], axis=1) * dinv
        t = jnp.maximum(
            jnp.dot(a, w1_ref[...], preferred_element_type=jnp.float32)
            + b1_ref[...], 0.0)
        z = jnp.dot(t, w2_ref[...], preferred_element_type=jnp.float32) * dinv
        for k, o in enumerate(outs):
            o[...] = z[:, k * 128:(k + 1) * 128]

    return pl.pallas_call(
        body,
        grid=(_NBLK,),
        in_specs=[pl.BlockSpec((_BR, 128), lambda i: (i, 0))] * 2 + [
            _DEG_SPEC,
            _full_spec((_DIN, _H)),
            _full_spec((1, _H)),
            _full_spec((_H, _H)),
        ],
        out_specs=[pl.BlockSpec((_BR, 128), lambda i: (i, 0))] * 4,
        out_shape=[jax.ShapeDtypeStruct((_NP, 128), jnp.float32)] * 4,
    )(y0[0], y0[1], deg2, W1, b1, W2)


def _mm3_call(y1, deg2, b2, W3):
    def body(yc0, yc1, yc2, yc3, deg_ref, b2_ref, w3_ref, *outs):
        dinv = _dinv_from(deg_ref[...])
        h = jnp.concatenate([yc0[...], yc1[...], yc2[...], yc3[...]], axis=1)
        t = jnp.maximum(h * dinv + b2_ref[...], 0.0)
        z = jnp.dot(t, w3_ref[...], preferred_element_type=jnp.float32) * dinv
        for k, o in enumerate(outs):
            o[...] = z[:, k * 128:(k + 1) * 128]

    return pl.pallas_call(
        body,
        grid=(_NBLK,),
        in_specs=[pl.BlockSpec((_BR, 128), lambda i: (i, 0))] * 4 + [
            _DEG_SPEC,
            _full_spec((1, _H)),
            _full_spec((_H, _H)),
        ],
        out_specs=[pl.BlockSpec((_BR, 128), lambda i: (i, 0))] * 4,
        out_shape=[jax.ShapeDtypeStruct((_NP, 128), jnp.float32)] * 4,
    )(*y1, deg2, b2, W3)


def _final_call(y2, deg2, b3, batchp, Wlin, blin):
    def body(yc0, yc1, yc2, yc3, deg_ref, b3_ref, bat_ref, wl_ref, bl_ref,
             out_ref, sacc, cacc):
        i = pl.program_id(0)

        @pl.when(i == 0)
        def _():
            sacc[...] = jnp.zeros_like(sacc)
            cacc[...] = jnp.zeros_like(cacc)

        dinv = _dinv_from(deg_ref[...])
        h = jnp.concatenate([yc0[...], yc1[...], yc2[...], yc3[...]], axis=1)
        h3 = h * dinv + b3_ref[...]
        gio = lax.broadcasted_iota(jnp.int32, (_G, _BR), 0)
        oh = (bat_ref[...] == gio).astype(jnp.float32)
        sacc[...] += jnp.dot(oh, h3, preferred_element_type=jnp.float32)
        cacc[...] += jnp.sum(oh, axis=1, keepdims=True)

        @pl.when(i == _NBLK - 1)
        def _():
            pooled = sacc[...] / jnp.maximum(cacc[...][:, 0:1], 1.0)
            out_ref[...] = (
                jnp.dot(pooled, wl_ref[...], preferred_element_type=jnp.float32)
                + bl_ref[...])

    return pl.pallas_call(
        body,
        grid=(_NBLK,),
        in_specs=[pl.BlockSpec((_BR, 128), lambda i: (i, 0))] * 4 + [
            _DEG_SPEC,
            _full_spec((1, _H)),
            pl.BlockSpec((1, _BR), lambda i: (0, i)),
            _full_spec((_H, _DOUT)),
            _full_spec((1, _DOUT)),
        ],
        out_specs=pl.BlockSpec((_G, _DOUT), lambda i: (0, 0)),
        out_shape=jax.ShapeDtypeStruct((_G, _DOUT), jnp.float32),
        scratch_shapes=[
            pltpu.VMEM((_G, _H), jnp.float32),
            pltpu.VMEM((_G, 128), jnp.float32),
        ],
    )(*y2, deg2, b3, batchp, Wlin, blin)


def kernel(x, edge_index, batch, W1, b1, W2, b2, W3, b3, Wlin, blin):
    f32 = jnp.float32
    x_pad = jnp.concatenate([x, jnp.zeros((_NP - _N, _DIN), f32)], axis=0)
    pad_idx = jnp.full((_PE - _E,), _N, jnp.int32)
    srcp = jnp.concatenate([edge_index[0], pad_idx])
    dstp = jnp.concatenate([edge_index[1], pad_idx])
    src3 = srcp.reshape(_NS, _EB, 128)
    dst3s = dstp.reshape(_NS, _EB, 128)
    dst3 = dstp.reshape(_NC * _NS, _DB, 128)
    batchp = jnp.concatenate(
        [batch, jnp.full((_NP - _N,), _G, jnp.int32)]).reshape(1, _NP)
    zeros16 = jnp.zeros((_NP, 16), f32)
    ones16 = jnp.ones((128, 16), f32)
    b1r = b1.reshape(1, _H)
    b2r = b2.reshape(1, _H)
    b3r = b3.reshape(1, _H)
    blinr = blin.reshape(1, _DOUT)

    deg2 = _deg_call(dst3, zeros16, ones16)
    z0 = _prescale_call(x_pad, deg2)
    y0 = _scatter_call(z0, src3, dst3s)
    z1 = _mm12_call(y0, deg2, W1, b1r, W2)
    y1 = _scatter_call(z1, src3, dst3s)
    z2 = _mm3_call(y1, deg2, b2r, W3)
    y2 = _scatter_call(z2, src3, dst3s)
    return _final_call(y2, deg2, b3r, batchp, Wlin, blinr)


# speedup vs baseline: 6.7742x; 1.0366x over previous
import functools

import jax
import jax.numpy as jnp
from jax import lax
from jax.experimental import pallas as pl
from jax.experimental.pallas import tpu as pltpu
from jax.experimental.pallas import tpu_sc as plsc

_N = 10000
_E = 160000
_DIN = 256
_H = 512
_G = 64
_DOUT = 500

_NP = 10240
_PE = 163840
_NS = 16
_NC = 2
_EB = _PE // (_NS * 128)
_DB = _PE // (_NC * _NS * 128)
_RT = _NP // _NS
_BR = 512
_NBLK = _NP // _BR


def _sc_mesh():
    return plsc.VectorSubcoreMesh(core_axis_name="c", subcore_axis_name="s")


def _deg_call(dst3, zeros16, ones16):
    @functools.partial(
        pl.kernel,
        out_type=jax.ShapeDtypeStruct((_NC, _NP, 16), jnp.float32),
        mesh=_sc_mesh(),
        scratch_types=[
            pltpu.VMEM((_DB, 128), jnp.int32),
            pltpu.VMEM((128, 16), jnp.float32),
            pltpu.VMEM_SHARED((_NP, 16), jnp.float32),
        ],
    )
    def deg_kernel(dst_hbm, z_hbm, o_hbm, out_hbm, idx_v, ones_v, acc):
        c = lax.axis_index("c")
        s = lax.axis_index("s")
        wid = c * _NS + s
        pltpu.sync_copy(dst_hbm.at[wid], idx_v)
        pltpu.sync_copy(o_hbm, ones_v)
        pltpu.sync_copy(z_hbm.at[pl.ds(s * _RT, _RT)], acc.at[pl.ds(s * _RT, _RT)])
        plsc.subcore_barrier()

        def step(j, carry):
            pltpu.sync_copy(ones_v, acc.at[idx_v.at[j]], add=True)
            return carry

        lax.fori_loop(0, _DB, step, 0)
        plsc.subcore_barrier()
        pltpu.sync_copy(acc.at[pl.ds(s * _RT, _RT)],
                        out_hbm.at[c, pl.ds(s * _RT, _RT)])

    return deg_kernel(dst3, zeros16, ones16)


def _scatter_call(z_chunks, src3, dst3s):
    nc = len(z_chunks)
    nc_half = nc // 2
    eb = _PE // (_NS * 128)
    nph = 5
    nbuf = 2
    hb = eb // nph

    @functools.partial(
        pl.kernel,
        out_type=[jax.ShapeDtypeStruct((_NP, 128), jnp.float32)] * nc,
        mesh=_sc_mesh(),
        scratch_types=[
            pltpu.VMEM((hb, 128), jnp.int32),
            pltpu.VMEM((hb, 128), jnp.int32),
            pltpu.VMEM((128, 128), jnp.float32),
            pltpu.VMEM((128, 128), jnp.float32),
            pltpu.VMEM_SHARED((_NP, 128), jnp.float32),
            pltpu.SemaphoreType.DMA,
            pltpu.SemaphoreType.DMA,
        ],
    )
    def scat_kernel(*refs):
        z_refs = refs[:nc]
        src_hbm = refs[nc]
        dst_hbm = refs[nc + 1]
        y_refs = refs[nc + 2:2 * nc + 2]
        src_idx, dst_idx = refs[2 * nc + 2:2 * nc + 4]
        gbufs = refs[2 * nc + 4:2 * nc + 6]
        acc = refs[2 * nc + 6]
        sems = refs[2 * nc + 7:2 * nc + 9]
        c = lax.axis_index("c")
        s = lax.axis_index("s")

        for chunk in range(nc):
            z_ref = z_refs[chunk]
            y_ref = y_refs[chunk]

            @pl.when(c == chunk // nc_half)
            def _():
                pltpu.sync_copy(z_ref.at[pl.ds(s * _RT, _RT)],
                                acc.at[pl.ds(s * _RT, _RT)])
                plsc.subcore_barrier()

                for ph in range(nph):
                    pltpu.sync_copy(src_hbm.at[s, pl.ds(ph * hb, hb)], src_idx)
                    pltpu.sync_copy(dst_hbm.at[s, pl.ds(ph * hb, hb)], dst_idx)

                    def step(i, carry):
                        b = nbuf * i
                        descs = [
                            pltpu.async_copy(z_ref.at[src_idx.at[b + k]],
                                             gbufs[k], sems[k])
                            for k in range(nbuf)
                        ]
                        for k in range(nbuf):
                            descs[k].wait()
                            pltpu.sync_copy(gbufs[k],
                                            acc.at[dst_idx.at[b + k]],
                                            add=True)
                        return carry

                    lax.fori_loop(0, hb // nbuf, step, 0)
                plsc.subcore_barrier()
                pltpu.sync_copy(acc.at[pl.ds(s * _RT, _RT)],
                                y_ref.at[pl.ds(s * _RT, _RT)])
                plsc.subcore_barrier()

    return scat_kernel(*z_chunks, src3, dst3s)


def _dinv_from(deg_blk):
    cnt = deg_blk[0, :, 0:1] + deg_blk[1, :, 0:1]
    return lax.rsqrt(1.0 + cnt)


_DEG_SPEC = pl.BlockSpec((_NC, _BR, 16), lambda i: (0, i, 0))


def _full_spec(shape):
    return pl.BlockSpec(shape, lambda i: tuple(0 for _ in shape))


def _prescale_call(x_pad, deg2):
    def body(x_ref, deg_ref, o0_ref, o1_ref):
        z = x_ref[...] * _dinv_from(deg_ref[...])
        o0_ref[...] = z[:, :128]
        o1_ref[...] = z[:, 128:]

    return pl.pallas_call(
        body,
        grid=(_NBLK,),
        in_specs=[pl.BlockSpec((_BR, _DIN), lambda i: (i, 0)), _DEG_SPEC],
        out_specs=[pl.BlockSpec((_BR, 128), lambda i: (i, 0))] * 2,
        out_shape=[jax.ShapeDtypeStruct((_NP, 128), jnp.float32)] * 2,
    )(x_pad, deg2)


def _mm12_call(y0, deg2, W1, b1, W2):
    def body(yc0, yc1, deg_ref, w1_ref, b1_ref, w2_ref, *outs):
        dinv = _dinv_from(deg_ref[...])
        a = jnp.concatenate([yc0[...], yc1[...]], axis=1) * dinv
        t = jnp.maximum(
            jnp.dot(a, w1_ref[...], preferred_element_type=jnp.float32)
            + b1_ref[...], 0.0)
        z = jnp.dot(t, w2_ref[...], preferred_element_type=jnp.float32) * dinv
        for k, o in enumerate(outs):
            o[...] = z[:, k * 128:(k + 1) * 128]

    return pl.pallas_call(
        body,
        grid=(_NBLK,),
        in_specs=[pl.BlockSpec((_BR, 128), lambda i: (i, 0))] * 2 + [
            _DEG_SPEC,
            _full_spec((_DIN, _H)),
            _full_spec((1, _H)),
            _full_spec((_H, _H)),
        ],
        out_specs=[pl.BlockSpec((_BR, 128), lambda i: (i, 0))] * 4,
        out_shape=[jax.ShapeDtypeStruct((_NP, 128), jnp.float32)] * 4,
    )(y0[0], y0[1], deg2, W1, b1, W2)


def _mm3_call(y1, deg2, b2, W3):
    def body(yc0, yc1, yc2, yc3, deg_ref, b2_ref, w3_ref, *outs):
        dinv = _dinv_from(deg_ref[...])
        h = jnp.concatenate([yc0[...], yc1[...], yc2[...], yc3[...]], axis=1)
        t = jnp.maximum(h * dinv + b2_ref[...], 0.0)
        z = jnp.dot(t, w3_ref[...], preferred_element_type=jnp.float32) * dinv
        for k, o in enumerate(outs):
            o[...] = z[:, k * 128:(k + 1) * 128]

    return pl.pallas_call(
        body,
        grid=(_NBLK,),
        in_specs=[pl.BlockSpec((_BR, 128), lambda i: (i, 0))] * 4 + [
            _DEG_SPEC,
            _full_spec((1, _H)),
            _full_spec((_H, _H)),
        ],
        out_specs=[pl.BlockSpec((_BR, 128), lambda i: (i, 0))] * 4,
        out_shape=[jax.ShapeDtypeStruct((_NP, 128), jnp.float32)] * 4,
    )(*y1, deg2, b2, W3)


def _final_call(y2, deg2, b3, batchp, Wlin, blin):
    def body(yc0, yc1, yc2, yc3, deg_ref, b3_ref, bat_ref, wl_ref, bl_ref,
             out_ref, sacc, cacc):
        i = pl.program_id(0)

        @pl.when(i == 0)
        def _():
            sacc[...] = jnp.zeros_like(sacc)
            cacc[...] = jnp.zeros_like(cacc)

        dinv = _dinv_from(deg_ref[...])
        h = jnp.concatenate([yc0[...], yc1[...], yc2[...], yc3[...]], axis=1)
        h3 = h * dinv + b3_ref[...]
        gio = lax.broadcasted_iota(jnp.int32, (_G, _BR), 0)
        oh = (bat_ref[...] == gio).astype(jnp.float32)
        sacc[...] += jnp.dot(oh, h3, preferred_element_type=jnp.float32)
        cacc[...] += jnp.sum(oh, axis=1, keepdims=True)

        @pl.when(i == _NBLK - 1)
        def _():
            pooled = sacc[...] / jnp.maximum(cacc[...][:, 0:1], 1.0)
            out_ref[...] = (
                jnp.dot(pooled, wl_ref[...], preferred_element_type=jnp.float32)
                + bl_ref[...])

    return pl.pallas_call(
        body,
        grid=(_NBLK,),
        in_specs=[pl.BlockSpec((_BR, 128), lambda i: (i, 0))] * 4 + [
            _DEG_SPEC,
            _full_spec((1, _H)),
            pl.BlockSpec((1, _BR), lambda i: (0, i)),
            _full_spec((_H, _DOUT)),
            _full_spec((1, _DOUT)),
        ],
        out_specs=pl.BlockSpec((_G, _DOUT), lambda i: (0, 0)),
        out_shape=jax.ShapeDtypeStruct((_G, _DOUT), jnp.float32),
        scratch_shapes=[
            pltpu.VMEM((_G, _H), jnp.float32),
            pltpu.VMEM((_G, 128), jnp.float32),
        ],
    )(*y2, deg2, b3, batchp, Wlin, blin)


def kernel(x, edge_index, batch, W1, b1, W2, b2, W3, b3, Wlin, blin):
    f32 = jnp.float32
    x_pad = jnp.concatenate([x, jnp.zeros((_NP - _N, _DIN), f32)], axis=0)
    pad_idx = jnp.full((_PE - _E,), _N, jnp.int32)
    srcp = jnp.concatenate([edge_index[0], pad_idx])
    dstp = jnp.concatenate([edge_index[1], pad_idx])
    src3 = srcp.reshape(_NS, _PE // (_NS * 128), 128)
    dst3s = dstp.reshape(_NS, _PE // (_NS * 128), 128)
    dst3 = dstp.reshape(_NC * _NS, _DB, 128)
    batchp = jnp.concatenate(
        [batch, jnp.full((_NP - _N,), _G, jnp.int32)]).reshape(1, _NP)
    zeros16 = jnp.zeros((_NP, 16), f32)
    ones16 = jnp.ones((128, 16), f32)
    b1r = b1.reshape(1, _H)
    b2r = b2.reshape(1, _H)
    b3r = b3.reshape(1, _H)
    blinr = blin.reshape(1, _DOUT)

    deg2 = _deg_call(dst3, zeros16, ones16)
    z0 = _prescale_call(x_pad, deg2)
    y0 = _scatter_call(z0, src3, dst3s)
    z1 = _mm12_call(y0, deg2, W1, b1r, W2)
    y1 = _scatter_call(z1, src3, dst3s)
    z2 = _mm3_call(y1, deg2, b2r, W3)
    y2 = _scatter_call(z2, src3, dst3s)
    return _final_call(y2, deg2, b3r, batchp, Wlin, blinr)
